# Initial kernel scaffold; baseline (speedup 1.0000x reference)
import jax
import jax.numpy as jnp
from jax.experimental import pallas as pl


def kernel(x, alpha, lin_w, lin_b, conv1_w, conv1_b, conv2_w, conv2_b, edge_index):
    raise NotImplementedError("write your pallas kernel here")


# speedup vs baseline: 14.2644x; 14.2644x over previous
import functools

import jax
import jax.numpy as jnp
from jax import lax
from jax.experimental import pallas as pl
from jax.experimental.pallas import tpu as pltpu
from jax.experimental.pallas import tpu_sc as plsc


def _cdiv(a, b):
    return (a + b - 1) // b


def _make_pi_kernel(n_nodes, e1, e1p, pi_iters):
    npad = _cdiv(n_nodes, 1024) * 1024
    per_tile = e1p // 16
    t_chunks = per_tile // 128
    c_chunks = per_tile // 16
    slice_sz = npad // 16
    sum_chunks = n_nodes // 16
    mesh = plsc.VectorSubcoreMesh(core_axis_name="c", subcore_axis_name="s")

    @functools.partial(
        pl.kernel,
        out_type=jax.ShapeDtypeStruct((16, t_chunks, 128), jnp.float32),
        mesh=mesh,
        compiler_params=pltpu.CompilerParams(needs_layout_passes=False),
        scratch_types=[
            pltpu.VMEM((t_chunks, 128), jnp.int32),
            pltpu.VMEM((t_chunks, 128), jnp.int32),
            pltpu.VMEM((t_chunks, 128), jnp.float32),
            pltpu.VMEM((t_chunks, 128), jnp.float32),
            pltpu.VMEM((npad,), jnp.float32),
            pltpu.VMEM((npad,), jnp.float32),
            pltpu.VMEM((npad,), jnp.float32),
            pltpu.VMEM((slice_sz,), jnp.float32),
            pltpu.VMEM((slice_sz,), jnp.float32),
            pltpu.VMEM((16,), jnp.float32),
            pltpu.VMEM_SHARED((16 * npad,), jnp.float32),
            pltpu.VMEM_SHARED((npad,), jnp.float32),
        ],
    )
    def pi_kernel(row_h, col_h, alpha_h, ew_h,
                  row_v, col_v, p_v, val_v, pi_v, lacc_v, r_v, tmp_v, red_v,
                  alpha_v, acc_all, pi_sh):
        c = lax.axis_index("c")
        w = lax.axis_index("s")
        iota16 = lax.broadcasted_iota(jnp.int32, (16,), 0)
        zero16 = jnp.zeros((16,), jnp.float32)

        pltpu.sync_copy(row_h.at[w], row_v)
        pltpu.sync_copy(col_h.at[w], col_v)
        pltpu.sync_copy(alpha_h, alpha_v)
        va = alpha_v[...]

        def zero_lacc(_i, carry):
            lacc_v[pl.ds(_i * 16, 16)] = zero16
            return carry

        def allreduce_to_pi():
            pltpu.sync_copy(lacc_v, acc_all.at[pl.ds(w * npad, npad)])
            plsc.subcore_barrier()

            def zred(_i, carry):
                red_v[pl.ds(_i * 16, 16)] = zero16
                return carry
            lax.fori_loop(0, slice_sz // 16, zred, 0)

            def red_t(t, carry):
                pltpu.sync_copy(
                    acc_all.at[pl.ds(t * npad + w * slice_sz, slice_sz)], tmp_v)

                def addc(i, cc):
                    red_v[pl.ds(i * 16, 16)] = (
                        red_v[pl.ds(i * 16, 16)] + tmp_v[pl.ds(i * 16, 16)])
                    return cc
                return lax.fori_loop(0, slice_sz // 16, addc, carry)
            lax.fori_loop(0, 16, red_t, 0)
            pltpu.sync_copy(red_v, pi_sh.at[pl.ds(w * slice_sz, slice_sz)])
            plsc.subcore_barrier()
            pltpu.sync_copy(pi_sh, pi_v)

        lax.fori_loop(0, npad // 16, zero_lacc, 0)

        def deg_body(b, carry):
            j = b // 8
            k = (b % 8) * 16
            gid = w * per_tile + b * 16 + iota16
            ind = jnp.where(gid < e1, 1.0, 0.0).astype(jnp.float32)
            idx = row_v[j, pl.ds(k, 16)]
            plsc.addupdate_scatter(lacc_v, [idx], ind)
            val_v[j, pl.ds(k, 16)] = ind
            return carry
        lax.fori_loop(0, c_chunks, deg_body, 0)
        allreduce_to_pi()

        def p_body(b, carry):
            j = b // 8
            k = (b % 8) * 16
            idx = row_v[j, pl.ds(k, 16)]
            dg = plsc.load_gather(pi_v, [idx])
            p_v[j, pl.ds(k, 16)] = val_v[j, pl.ds(k, 16)] / dg
            return carry
        lax.fori_loop(0, c_chunks, p_body, 0)

        inv_n = jnp.float32(1.0 / n_nodes)

        def init_body(i, carry):
            pi_v[pl.ds(i * 16, 16)] = jnp.full((16,), inv_n, jnp.float32)
            return carry
        lax.fori_loop(0, npad // 16, init_body, 0)

        def iter_body(_t, carry):
            lax.fori_loop(0, npad // 16, zero_lacc, 0)

            def vb(b, cc):
                j = b // 8
                k = (b % 8) * 16
                idx = row_v[j, pl.ds(k, 16)]
                g = plsc.load_gather(pi_v, [idx])
                v = g * p_v[j, pl.ds(k, 16)]
                cidx = col_v[j, pl.ds(k, 16)]
                plsc.addupdate_scatter(lacc_v, [cidx], v)
                return cc
            lax.fori_loop(0, c_chunks, vb, 0)
            allreduce_to_pi()

            def ab(i, acc16):
                v = pi_v[pl.ds(i * 16, 16)]
                v2 = (1.0 - va) * v + va * inv_n
                pi_v[pl.ds(i * 16, 16)] = v2
                return acc16 + v2
            s16 = lax.fori_loop(0, sum_chunks, ab, zero16)
            total = jnp.sum(s16)

            def nb(i, cc):
                pi_v[pl.ds(i * 16, 16)] = pi_v[pl.ds(i * 16, 16)] / total
                return cc
            lax.fori_loop(0, sum_chunks, nb, 0)
            return carry
        lax.fori_loop(0, pi_iters, iter_body, 0)

        def rb(i, carry):
            v = pi_v[pl.ds(i * 16, 16)]
            x = jnp.maximum(v, 1e-12)
            ii = plsc.bitcast(x, jnp.int32)
            ii = jnp.int32(0x5F3759DF) - (ii >> 1)
            y = plsc.bitcast(ii, jnp.float32)
            y = y * (1.5 - 0.5 * x * y * y)
            y = y * (1.5 - 0.5 * x * y * y)
            y = y * (1.5 - 0.5 * x * y * y)
            r_v[pl.ds(i * 16, 16)] = y
            pi_v[pl.ds(i * 16, 16)] = x * y
            return carry
        lax.fori_loop(0, npad // 16, rb, 0)

        def wb(b, carry):
            j = b // 8
            k = (b % 8) * 16
            ridx = row_v[j, pl.ds(k, 16)]
            cidx = col_v[j, pl.ds(k, 16)]
            ps = plsc.load_gather(pi_v, [ridx])
            rc = plsc.load_gather(r_v, [cidx])
            val_v[j, pl.ds(k, 16)] = 0.5 * p_v[j, pl.ds(k, 16)] * ps * rc
            return carry
        lax.fori_loop(0, c_chunks, wb, 0)

        @pl.when(c == 0)
        def _():
            pltpu.sync_copy(val_v, ew_h.at[w])

    return pi_kernel


def _make_prop_kernel(n_nodes, d, e2p):
    per_tile = e2p // 32
    t_chunks = per_tile // 128
    npad = _cdiv(n_nodes, 1024) * 1024
    rows_per_tile = npad // 16
    zrows = 128
    mesh = plsc.VectorSubcoreMesh(core_axis_name="c", subcore_axis_name="s")

    @functools.partial(
        pl.kernel,
        out_type=jax.ShapeDtypeStruct((2 * npad, d), jnp.float32),
        mesh=mesh,
        compiler_params=pltpu.CompilerParams(needs_layout_passes=False),
        scratch_types=[
            pltpu.VMEM((128,), jnp.int32),
            pltpu.VMEM((128,), jnp.int32),
            pltpu.VMEM((128,), jnp.float32),
            pltpu.VMEM((128, d), jnp.float32),
            pltpu.VMEM_SHARED((npad, d), jnp.float32),
            pltpu.SemaphoreType.DMA,
        ],
    )
    def prop_kernel(src_h, dst_h, ew_h, h_h, out_h,
                    sidx_v, didx_v, ewc_v, rows_v, acc, sem):
        c = lax.axis_index("c")
        s = lax.axis_index("s")
        g = c * 16 + s
        zero16 = jnp.zeros((16,), jnp.float32)

        def zb(r, carry):
            for i in range(d // 16):
                rows_v[r, pl.ds(i * 16, 16)] = zero16
            return carry
        lax.fori_loop(0, 128, zb, 0)
        row0 = s * rows_per_tile
        for t in range(rows_per_tile // zrows):
            pltpu.sync_copy(rows_v, acc.at[pl.ds(row0 + t * zrows, zrows), :])
        plsc.subcore_barrier()

        def chunk_body(j, carry):
            pltpu.sync_copy(src_h.at[g].at[j], sidx_v)
            pltpu.sync_copy(dst_h.at[g].at[j], didx_v)
            pltpu.sync_copy(ew_h.at[g].at[j], ewc_v)
            pltpu.async_copy(h_h.at[sidx_v], rows_v, sem).wait()

            def scale_g(gi, cc):
                ew16 = ewc_v[pl.ds(gi * 16, 16)]
                for rr in range(16):
                    e = ew16[rr]
                    r = gi * 16 + rr
                    for kk in range(d // 16):
                        rows_v[r, pl.ds(kk * 16, 16)] = (
                            rows_v[r, pl.ds(kk * 16, 16)] * e)
                return cc
            lax.fori_loop(0, 8, scale_g, 0)
            pltpu.async_copy(rows_v, acc.at[didx_v], sem, add=True).wait()
            return carry
        lax.fori_loop(0, t_chunks, chunk_body, 0)
        plsc.subcore_barrier()

        pltpu.sync_copy(
            acc.at[pl.ds(row0, rows_per_tile), :],
            out_h.at[pl.ds(c * npad + row0, rows_per_tile), :])

    return prop_kernel


_BLK = 1000


def _mm3_body(x_ref, w1_ref, w2_ref, wl_ref, bl_ref, h1_ref, h2_ref, o0_ref):
    xb = x_ref[...]
    h1_ref[...] = lax.dot_general(
        xb, w1_ref[...], (((1,), (0,)), ((), ())),
        preferred_element_type=jnp.float32)
    h2_ref[...] = lax.dot_general(
        xb, w2_ref[...], (((1,), (0,)), ((), ())),
        preferred_element_type=jnp.float32)
    o0_ref[...] = lax.dot_general(
        xb, wl_ref[...], (((1,), (1,)), ((), ())),
        preferred_element_type=jnp.float32) + bl_ref[...]


def _mm3(x, w1, w2, wl, bl):
    n, d = x.shape
    grid = n // _BLK
    return pl.pallas_call(
        _mm3_body,
        grid=(grid,),
        in_specs=[
            pl.BlockSpec((_BLK, d), lambda i: (i, 0)),
            pl.BlockSpec((d, d), lambda i: (0, 0)),
            pl.BlockSpec((d, d), lambda i: (0, 0)),
            pl.BlockSpec((d, d), lambda i: (0, 0)),
            pl.BlockSpec((1, d), lambda i: (0, 0)),
        ],
        out_specs=[
            pl.BlockSpec((_BLK, d), lambda i: (i, 0)),
            pl.BlockSpec((_BLK, d), lambda i: (i, 0)),
            pl.BlockSpec((_BLK, d), lambda i: (i, 0)),
        ],
        out_shape=[jax.ShapeDtypeStruct((n, d), jnp.float32)] * 3,
    )(x, w1, w2, wl, bl)


def _add2_body(a_ref, b_ref, o_ref):
    o_ref[...] = a_ref[0] + b_ref[0]


def _add2(y2, n):
    _, npad, d = y2.shape
    grid = n // _BLK
    return pl.pallas_call(
        _add2_body,
        grid=(grid,),
        in_specs=[
            pl.BlockSpec((1, _BLK, d), lambda i: (0, i, 0)),
            pl.BlockSpec((1, _BLK, d), lambda i: (1, i, 0)),
        ],
        out_specs=pl.BlockSpec((_BLK, d), lambda i: (i, 0)),
        out_shape=jax.ShapeDtypeStruct((n, d), jnp.float32),
    )(y2, y2)


def _combine_body(o0_ref, y1a_ref, y1b_ref, y2a_ref, y2b_ref, c1_ref, c2_ref,
                  o_ref):
    o_ref[...] = (o0_ref[...] + y1a_ref[0] + y1b_ref[0]
                  + y2a_ref[0] + y2b_ref[0] + c1_ref[...] + c2_ref[...])


def _combine(o0, y1, y2, c1, c2):
    n, d = o0.shape
    grid = n // _BLK
    return pl.pallas_call(
        _combine_body,
        grid=(grid,),
        in_specs=[
            pl.BlockSpec((_BLK, d), lambda i: (i, 0)),
            pl.BlockSpec((1, _BLK, d), lambda i: (0, i, 0)),
            pl.BlockSpec((1, _BLK, d), lambda i: (1, i, 0)),
            pl.BlockSpec((1, _BLK, d), lambda i: (0, i, 0)),
            pl.BlockSpec((1, _BLK, d), lambda i: (1, i, 0)),
            pl.BlockSpec((1, d), lambda i: (0, 0)),
            pl.BlockSpec((1, d), lambda i: (0, 0)),
        ],
        out_specs=pl.BlockSpec((_BLK, d), lambda i: (i, 0)),
        out_shape=jax.ShapeDtypeStruct((n, d), jnp.float32),
    )(o0, y1, y1, y2, y2, c1, c2)


def kernel(x, alpha, lin_w, lin_b, conv1_w, conv1_b, conv2_w, conv2_b,
           edge_index):
    n, d = x.shape
    e = edge_index.shape[1]
    e1 = e + n
    e1p = _cdiv(e1, 16 * 128) * (16 * 128)
    e2 = 2 * e1
    e2p = _cdiv(e2, 32 * 128) * (32 * 128)
    pi_iters = 20

    loops = jnp.arange(n, dtype=jnp.int32)
    row = jnp.concatenate([edge_index[0], loops])
    col = jnp.concatenate([edge_index[1], loops])
    row_p = jnp.pad(row, (0, e1p - e1)).reshape(16, e1p // 16 // 128, 128)
    col_p = jnp.pad(col, (0, e1p - e1)).reshape(16, e1p // 16 // 128, 128)
    alpha16 = jnp.full((16,), alpha, jnp.float32)

    pi_kernel = _make_pi_kernel(n, e1, e1p, pi_iters)
    ew_half = pi_kernel(row_p, col_p, alpha16).reshape(-1)[:e1]

    src = jnp.concatenate([row, col])
    dst = jnp.concatenate([col, row])
    eww = jnp.concatenate([ew_half, ew_half])
    shp = (32, e2p // 32 // 128, 128)
    src_p = jnp.pad(src, (0, e2p - e2)).reshape(shp)
    dst_p = jnp.pad(dst, (0, e2p - e2)).reshape(shp)
    ew_p = jnp.pad(eww, (0, e2p - e2)).reshape(shp)

    prop = _make_prop_kernel(n, d, e2p)
    c1 = conv1_b.reshape(1, d)
    c2 = conv2_b.reshape(1, d)
    bl = lin_b.reshape(1, d)

    npad = _cdiv(n, 1024) * 1024
    xc = x
    for _ in range(2):
        h1, h2, o0 = _mm3(xc, conv1_w, conv2_w, lin_w, bl)
        y1 = prop(src_p, dst_p, ew_p, h1).reshape(2, npad, d)
        t2 = prop(src_p, dst_p, ew_p, h2).reshape(2, npad, d)
        t = _add2(t2, n)
        y2 = prop(src_p, dst_p, ew_p, t).reshape(2, npad, d)
        xc = _combine(o0, y1, y2, c1, c2)
    return xc
